# Initial kernel scaffold; baseline (speedup 1.0000x reference)
import jax
import jax.numpy as jnp
from jax.experimental import pallas as pl


def kernel(token_ids, segment_ids, table):
    raise NotImplementedError("write your pallas kernel here")


# speedup vs baseline: 2.6183x; 2.6183x over previous
import functools

import jax
import jax.numpy as jnp
from jax import lax
from jax.experimental import pallas as pl
from jax.experimental.pallas import tpu as pltpu
from jax.experimental.pallas import tpu_sc as plsc

NUM_SENT = 8192
G = 128
GROUPS = 8
CHUNK = G * GROUPS


def _sc_gather_pool(token_ids_2d, segment_ids_2d, table):
    t_rows, g = token_ids_2d.shape
    assert g == G
    total = t_rows * G
    hidden = table.shape[1]

    info = plsc.get_sparse_core_info()
    nc, ns = info.num_cores, info.num_subcores
    nw = nc * ns
    assert total % (nw * CHUNK) == 0
    chunks_per_worker = total // (nw * CHUNK)
    rows_per_worker = t_rows // nw
    sent_per_tile = NUM_SENT // ns

    zeros_sum = jnp.zeros((sent_per_tile, hidden), jnp.float32)
    zeros_cnt = jnp.zeros((sent_per_tile, 16), jnp.float32)
    ones_blk = jnp.ones((G, 16), jnp.float32)

    mesh = plsc.VectorSubcoreMesh(core_axis_name="c", subcore_axis_name="s")

    @functools.partial(
        pl.kernel,
        mesh=mesh,
        compiler_params=pltpu.CompilerParams(use_tc_tiling_on_sc=False),
        out_type=[
            jax.ShapeDtypeStruct((total, hidden), jnp.float32),
            jax.ShapeDtypeStruct((nc * NUM_SENT, hidden), jnp.float32),
            jax.ShapeDtypeStruct((nc * NUM_SENT, 16), jnp.float32),
        ],
        scratch_types=[
            pltpu.VMEM((GROUPS, G), jnp.int32),
            pltpu.VMEM((GROUPS, G), jnp.int32),
            pltpu.VMEM((CHUNK, hidden), jnp.float32),
            pltpu.VMEM((G, 16), jnp.float32),
            pltpu.VMEM_SHARED((NUM_SENT, hidden), jnp.float32),
            pltpu.VMEM_SHARED((NUM_SENT, 16), jnp.float32),
            pltpu.SemaphoreType.DMA,
        ],
    )
    def body(tok_hbm, seg_hbm, table_hbm, zs_hbm, zc_hbm, ones_hbm,
             out_tok, out_psum, out_pcnt,
             idx_v, seg_v, rows_v, ones_v, sums_sp, cnts_sp, gsem):
        c = lax.axis_index("c")
        s = lax.axis_index("s")
        wid = s * nc + c

        pltpu.sync_copy(zs_hbm, sums_sp.at[pl.ds(s * sent_per_tile, sent_per_tile)])
        pltpu.sync_copy(zc_hbm, cnts_sp.at[pl.ds(s * sent_per_tile, sent_per_tile)])
        pltpu.sync_copy(ones_hbm, ones_v)
        plsc.subcore_barrier()

        row_base = wid * rows_per_worker

        def chunk_body(k, carry):
            r0 = row_base + k * GROUPS
            pltpu.sync_copy(tok_hbm.at[pl.ds(r0, GROUPS)], idx_v)
            pltpu.sync_copy(seg_hbm.at[pl.ds(r0, GROUPS)], seg_v)
            descs = []
            for j in range(GROUPS):
                descs.append(
                    pltpu.async_copy(
                        table_hbm.at[idx_v.at[j]],
                        rows_v.at[pl.ds(j * G, G)],
                        gsem,
                    )
                )
            for d in descs:
                d.wait()
            pltpu.sync_copy(rows_v, out_tok.at[pl.ds(r0 * G, CHUNK)])
            for j in range(GROUPS):
                pltpu.sync_copy(rows_v.at[pl.ds(j * G, G)],
                                sums_sp.at[seg_v.at[j]], add=True)
                pltpu.sync_copy(ones_v, cnts_sp.at[seg_v.at[j]], add=True)
            return carry

        lax.fori_loop(0, chunks_per_worker, chunk_body, 0)

        plsc.subcore_barrier()
        off = c * NUM_SENT + s * sent_per_tile
        pltpu.sync_copy(sums_sp.at[pl.ds(s * sent_per_tile, sent_per_tile)],
                        out_psum.at[pl.ds(off, sent_per_tile)])
        pltpu.sync_copy(cnts_sp.at[pl.ds(s * sent_per_tile, sent_per_tile)],
                        out_pcnt.at[pl.ds(off, sent_per_tile)])

    return body(token_ids_2d, segment_ids_2d, table, zeros_sum, zeros_cnt,
                ones_blk)


def _combine(psum, pcnt, hidden):
    def body(ps_ref, pc_ref, sent_ref, glob_ref):
        sums = ps_ref[0:NUM_SENT, :] + ps_ref[NUM_SENT:2 * NUM_SENT, :]
        cnts = pc_ref[0:NUM_SENT, 0:1] + pc_ref[NUM_SENT:2 * NUM_SENT, 0:1]
        sent = jnp.where(cnts > 0.0, sums / jnp.maximum(cnts, 1.0), 0.0)
        sent_ref[...] = sent
        glob_ref[...] = jnp.sum(sent, axis=0, keepdims=True) * (1.0 / NUM_SENT)

    return pl.pallas_call(
        body,
        out_shape=[
            jax.ShapeDtypeStruct((NUM_SENT, hidden), jnp.float32),
            jax.ShapeDtypeStruct((1, hidden), jnp.float32),
        ],
    )(psum, pcnt)


def kernel(token_ids, segment_ids, table):
    t = token_ids.shape[0]
    hidden = table.shape[1]
    tok2d = token_ids.astype(jnp.int32).reshape(t // G, G)
    seg2d = segment_ids.astype(jnp.int32).reshape(t // G, G)
    token_repr, psum, pcnt = _sc_gather_pool(tok2d, seg2d, table)
    sentence_repr, global_repr = _combine(psum, pcnt, hidden)
    return token_repr, sentence_repr, global_repr.reshape(hidden)
